# baseline (speedup 1.0000x reference)
import functools

import jax
import jax.numpy as jnp
from jax import lax
from jax.experimental import pallas as pl
from jax.experimental.pallas import tpu as pltpu
from jax.experimental.pallas import tpu_sc as plsc

EPS = 1e-5
Z_COEF = 0.001

S, D, H, E, F = 2048, 1024, 16, 8, 1536
DH = D // H
BM = 256
NBLK = 24
NBLK_PAD = 32
RPAD = NBLK * BM
NC, NS = 2, 16
NW = NC * NS
TPW = S // NW
D2 = D // 2


def _to_i32(a):
    return lax.bitcast_convert_type(a.reshape(a.shape[0], D2, 2), jnp.int32)


def _to_bf16(a):
    return lax.bitcast_convert_type(a, jnp.bfloat16).reshape(a.shape[0], D)


def _rmsnorm(v, w):
    return v * lax.rsqrt(jnp.mean(v * v, axis=-1, keepdims=True) + EPS) * w


def _fattn_body(x_ref, ln_ref, wq_ref, wk_ref, wv_ref, o_ref,
                h1_scr, k_scr, v_scr, *, scale, bq, hp):
    g = pl.program_id(0)
    i = pl.program_id(1)

    @pl.when(jnp.logical_and(g == 0, i == 0))
    def _():
        h1_scr[...] = _rmsnorm(x_ref[...], ln_ref[...]).astype(jnp.bfloat16)

    @pl.when(i == 0)
    def _():
        h1 = h1_scr[...]
        k_scr[...] = jnp.dot(h1, wk_ref[...],
                             preferred_element_type=jnp.float32).astype(
                                 jnp.bfloat16)
        v_scr[...] = jnp.dot(h1, wv_ref[...],
                             preferred_element_type=jnp.float32).astype(
                                 jnp.bfloat16)

    qg = jnp.dot(h1_scr[pl.ds(i * bq, bq), :], wq_ref[...],
                 preferred_element_type=jnp.float32).astype(jnp.bfloat16)
    k2 = k_scr[...]
    v2 = v_scr[...]

    outs = []
    for hh in range(hp):
        sl = slice(hh * DH, (hh + 1) * DH)
        s = lax.dot_general(qg[:, sl], k2[:, sl], (((1,), (1,)), ((), ())),
                            preferred_element_type=jnp.float32) * scale
        m = jnp.max(s, axis=-1, keepdims=True)
        p = jnp.exp(s - m)
        den = jnp.sum(p, axis=-1, keepdims=True)
        o = jnp.dot(p.astype(jnp.bfloat16), v2[:, sl],
                    preferred_element_type=jnp.float32) / den
        outs.append(o)
    o_ref[...] = jnp.concatenate(outs, axis=1).astype(jnp.bfloat16)


def _fused_attention(x, ln1_w, Wq, Wk, Wv, *, bq, hp):
    wspec = pl.BlockSpec((D, hp * DH), lambda g, i: (0, g))
    return pl.pallas_call(
        functools.partial(_fattn_body, scale=1.0 / (DH ** 0.5), bq=bq, hp=hp),
        grid=(H // hp, S // bq),
        in_specs=[
            pl.BlockSpec((S, D), lambda g, i: (0, 0)),
            pl.BlockSpec((1, D), lambda g, i: (0, 0)),
            wspec, wspec, wspec,
        ],
        out_specs=pl.BlockSpec((bq, hp * DH), lambda g, i: (i, g)),
        out_shape=jax.ShapeDtypeStruct((S, D), jnp.bfloat16),
        scratch_shapes=[
            pltpu.VMEM((S, D), jnp.bfloat16),
            pltpu.VMEM((S, hp * DH), jnp.bfloat16),
            pltpu.VMEM((S, hp * DH), jnp.bfloat16),
        ],
    )(x, ln1_w.reshape(1, D), Wq, Wk, Wv)


def _router_math(logits, w1_ref, w2_ref, d1_ref, d2_ref, be_ref, loss_ref):
    m = jnp.max(logits, axis=-1, keepdims=True)
    ex = jnp.exp(logits - m)
    den = jnp.sum(ex, axis=-1, keepdims=True)
    probs = ex / den

    cols = lax.broadcasted_iota(jnp.int32, (S, E), 1)
    i1 = jnp.argmax(probs, axis=-1)[:, None]
    w1 = jnp.max(probs, axis=-1, keepdims=True)
    oh1 = cols == i1
    masked = jnp.where(oh1, -jnp.inf, probs)
    i2 = jnp.argmax(masked, axis=-1)[:, None]
    w2 = jnp.max(masked, axis=-1, keepdims=True)
    oh2 = cols == i2
    tot = w1 + w2
    w1_ref[...] = w1 / tot
    w2_ref[...] = w2 / tot

    ind = jnp.logical_or(oh1, oh2).astype(jnp.bfloat16)

    C = 256
    r_io = lax.broadcasted_iota(jnp.int32, (C, C), 0)
    c_io = lax.broadcasted_iota(jnp.int32, (C, C), 1)
    tril = (c_io < r_io).astype(jnp.bfloat16)
    base = jnp.zeros((1, E), jnp.float32)
    chunks = []
    for i in range(S // C):
        ic = lax.slice(ind, (i * C, 0), ((i + 1) * C, E))
        chunks.append(jnp.dot(tril, ic, preferred_element_type=jnp.float32)
                      + base)
        base = base + jnp.sum(ic.astype(jnp.float32), axis=0, keepdims=True)
    pos = jnp.concatenate(chunks, axis=0)
    counts = base

    nbp = jnp.ceil(counts / BM) * BM
    e_r = lax.broadcasted_iota(jnp.int32, (E, E), 0)
    e_c = lax.broadcasted_iota(jnp.int32, (E, E), 1)
    triu = (e_r < e_c).astype(jnp.float32)
    off = jnp.dot(nbp, triu, preferred_element_type=jnp.float32)

    dest = off + pos
    d1_ref[...] = jnp.sum(jnp.where(oh1, dest, 0.0), axis=1,
                          keepdims=True).astype(jnp.int32)
    d2_ref[...] = jnp.sum(jnp.where(oh2, dest, 0.0), axis=1,
                          keepdims=True).astype(jnp.int32)

    bstart = lax.broadcasted_iota(jnp.int32, (NBLK_PAD, 1), 0).astype(
        jnp.float32) * BM
    inblk = jnp.logical_and(bstart >= off, bstart < off + nbp)
    ecols = lax.broadcasted_iota(jnp.int32, (NBLK_PAD, E), 1).astype(jnp.float32)
    be = jnp.sum(jnp.where(inblk, ecols + 1.0, 0.0), axis=1, keepdims=True) - 1.0
    be_ref[...] = be.astype(jnp.int32)

    z = jnp.log(den[:, 0]) + m[:, 0]
    z_loss = Z_COEF * jnp.mean(z * z)
    f = counts[0] / S
    P = jnp.mean(probs, axis=0)
    loss_ref[...] = (E * jnp.sum(f * P) + z_loss).reshape(1, 1)


def _postr_body(ao_ref, x_ref, wo_ref, ln_ref, wr_ref,
                x2_ref, h2_ref, w1_ref, w2_ref, d1_ref, d2_ref, be_ref,
                loss_ref, lg_scr, *, bm):
    i = pl.program_id(0)
    x2 = x_ref[...] + jnp.dot(ao_ref[...], wo_ref[...].astype(jnp.bfloat16),
                              preferred_element_type=jnp.float32)
    h2 = _rmsnorm(x2, ln_ref[...])
    x2_ref[...] = x2
    h2_ref[...] = h2
    lg_scr[pl.ds(i * bm, bm), :] = jnp.dot(
        h2, wr_ref[...], preferred_element_type=jnp.float32)

    @pl.when(i == S // bm - 1)
    def _():
        _router_math(lg_scr[...], w1_ref, w2_ref, d1_ref, d2_ref, be_ref,
                     loss_ref)


def _post_router(ao, x, Wo, ln2_w, Wr, *, bm):
    full = lambda i: (0, 0)
    return pl.pallas_call(
        functools.partial(_postr_body, bm=bm),
        grid=(S // bm,),
        in_specs=[
            pl.BlockSpec((bm, D), lambda i: (i, 0)),
            pl.BlockSpec((bm, D), lambda i: (i, 0)),
            pl.BlockSpec((D, D), full),
            pl.BlockSpec((1, D), full),
            pl.BlockSpec((D, E), full),
        ],
        out_specs=[
            pl.BlockSpec((bm, D), lambda i: (i, 0)),
            pl.BlockSpec((bm, D), lambda i: (i, 0)),
            pl.BlockSpec((S, 1), full),
            pl.BlockSpec((S, 1), full),
            pl.BlockSpec((S, 1), full),
            pl.BlockSpec((S, 1), full),
            pl.BlockSpec((NBLK_PAD, 1), full),
            pl.BlockSpec((1, 1), full),
        ],
        out_shape=[
            jax.ShapeDtypeStruct((S, D), jnp.float32),
            jax.ShapeDtypeStruct((S, D), jnp.float32),
            jax.ShapeDtypeStruct((S, 1), jnp.float32),
            jax.ShapeDtypeStruct((S, 1), jnp.float32),
            jax.ShapeDtypeStruct((S, 1), jnp.int32),
            jax.ShapeDtypeStruct((S, 1), jnp.int32),
            jax.ShapeDtypeStruct((NBLK_PAD, 1), jnp.int32),
            jax.ShapeDtypeStruct((1, 1), jnp.float32),
        ],
        scratch_shapes=[pltpu.VMEM((S, E), jnp.float32)],
    )(ao, x, Wo, ln2_w.reshape(1, D), Wr)


def _sc_mesh():
    return plsc.VectorSubcoreMesh(core_axis_name="c", subcore_axis_name="s")


def _sc_dispatch(h2, d1, d2):
    @functools.partial(
        pl.kernel,
        out_type=jax.ShapeDtypeStruct((RPAD, D), jnp.float32),
        mesh=_sc_mesh(),
        scratch_types=[
            pltpu.VMEM((TPW,), jnp.int32),
            pltpu.VMEM((TPW,), jnp.int32),
            pltpu.VMEM((TPW, D), jnp.float32),
            pltpu.SemaphoreType.DMA,
            pltpu.SemaphoreType.DMA,
        ],
    )
    def dispatch(h2_hbm, d1_hbm, d2_hbm, sorted_hbm,
                 idx1_v, idx2_v, rows_v, sem1, sem2):
        wid = lax.axis_index("s") * NC + lax.axis_index("c")
        base = wid * TPW
        pltpu.sync_copy(h2_hbm.at[pl.ds(base, TPW)], rows_v)
        pltpu.sync_copy(d1_hbm.at[pl.ds(base, TPW)], idx1_v)
        pltpu.sync_copy(d2_hbm.at[pl.ds(base, TPW)], idx2_v)
        cp1 = pltpu.async_copy(rows_v, sorted_hbm.at[idx1_v], sem1)
        cp2 = pltpu.async_copy(rows_v, sorted_hbm.at[idx2_v], sem2)
        cp1.wait()
        cp2.wait()

    return dispatch(h2, d1, d2)


def _moe_body(be_ref, x_ref, w1_ref, w2_ref, y_ref,
              w1b_scr, w2b_scr, prev_scr):
    b = pl.program_id(0)

    @pl.when(b == 0)
    def _():
        prev_scr[0] = -1

    be = be_ref[b]

    @pl.when(be >= 0)
    def _():
        @pl.when(prev_scr[0] != be)
        def _():
            w1b_scr[...] = w1_ref[0].astype(jnp.bfloat16)
            w2b_scr[...] = w2_ref[0].astype(jnp.bfloat16)
            prev_scr[0] = be

        xb = x_ref[...].astype(jnp.bfloat16)
        h = jnp.dot(xb, w1b_scr[...], preferred_element_type=jnp.float32)
        h = jax.nn.gelu(h).astype(jnp.bfloat16)
        y_ref[...] = jnp.dot(h, w2b_scr[...],
                             preferred_element_type=jnp.float32)


def _moe(be, sorted_x, W1, W2):
    grid_spec = pltpu.PrefetchScalarGridSpec(
        num_scalar_prefetch=1,
        grid=(NBLK,),
        in_specs=[
            pl.BlockSpec((BM, D), lambda b, be: (b, 0)),
            pl.BlockSpec((1, D, F), lambda b, be: (jnp.maximum(be[b], 0), 0, 0)),
            pl.BlockSpec((1, F, D), lambda b, be: (jnp.maximum(be[b], 0), 0, 0)),
        ],
        out_specs=pl.BlockSpec((BM, D), lambda b, be: (b, 0)),
        scratch_shapes=[
            pltpu.VMEM((D, F), jnp.bfloat16),
            pltpu.VMEM((F, D), jnp.bfloat16),
            pltpu.SMEM((1,), jnp.int32),
        ],
    )
    return pl.pallas_call(
        _moe_body,
        grid_spec=grid_spec,
        out_shape=jax.ShapeDtypeStruct((RPAD, D), jnp.float32),
    )(be, sorted_x, W1, W2)


def _sc_collect(y, d1, d2):
    @functools.partial(
        pl.kernel,
        out_type=(jax.ShapeDtypeStruct((S, D), jnp.float32),
                  jax.ShapeDtypeStruct((S, D), jnp.float32)),
        mesh=_sc_mesh(),
        scratch_types=[
            pltpu.VMEM((TPW,), jnp.int32),
            pltpu.VMEM((TPW,), jnp.int32),
            pltpu.VMEM((TPW, D), jnp.float32),
            pltpu.SemaphoreType.DMA,
        ],
    )
    def collect(y_hbm, d1_hbm, d2_hbm, y1_hbm, y2_hbm,
                idx1_v, idx2_v, rows_v, sem):
        wid = lax.axis_index("s") * NC + lax.axis_index("c")
        base = wid * TPW
        pltpu.sync_copy(d1_hbm.at[pl.ds(base, TPW)], idx1_v)
        pltpu.sync_copy(d2_hbm.at[pl.ds(base, TPW)], idx2_v)
        pltpu.async_copy(y_hbm.at[idx1_v], rows_v, sem).wait()
        pltpu.sync_copy(rows_v, y1_hbm.at[pl.ds(base, TPW)])
        pltpu.async_copy(y_hbm.at[idx2_v], rows_v, sem).wait()
        pltpu.sync_copy(rows_v, y2_hbm.at[pl.ds(base, TPW)])

    return collect(y, d1, d2)


def _combine_body(x2_ref, y1_ref, y2_ref, w1_ref, w2_ref, o_ref):
    o_ref[...] = (x2_ref[...] + w1_ref[...] * y1_ref[...]
                  + w2_ref[...] * y2_ref[...])


def _combine(x2, y1, y2, w1n, w2n, *, bm):
    return pl.pallas_call(
        _combine_body,
        grid=(S // bm,),
        in_specs=[
            pl.BlockSpec((bm, D), lambda i: (i, 0)),
            pl.BlockSpec((bm, D), lambda i: (i, 0)),
            pl.BlockSpec((bm, D), lambda i: (i, 0)),
            pl.BlockSpec((bm, 1), lambda i: (i, 0)),
            pl.BlockSpec((bm, 1), lambda i: (i, 0)),
        ],
        out_specs=pl.BlockSpec((bm, D), lambda i: (i, 0)),
        out_shape=jax.ShapeDtypeStruct((S, D), jnp.float32),
    )(x2, y1, y2, w1n, w2n)


def kernel(x, ln1_w, ln2_w, Wq, Wk, Wv, Wo, Wr, W1, W2):
    B = x.shape[0]
    xs = x.reshape(S, D)

    Wqb = Wq.astype(jnp.bfloat16)
    Wkb = Wk.astype(jnp.bfloat16)
    Wvb = Wv.astype(jnp.bfloat16)

    ao = _fused_attention(xs, ln1_w, Wqb, Wkb, Wvb, bq=512, hp=2)

    x2, h2, w1n, w2n, d1, d2, be, loss = _post_router(
        ao, xs, Wo, ln2_w, Wr, bm=256)
    d1r = d1.reshape(S)
    d2r = d2.reshape(S)

    sorted_x = _sc_dispatch(h2, d1r, d2r)
    y = _moe(be.reshape(NBLK_PAD), sorted_x, W1, W2)
    y1, y2 = _sc_collect(y, d1r, d2r)

    out = _combine(x2, y1, y2, w1n, w2n, bm=256)
    return (out.reshape(B, S, D), loss.reshape(()))


# speedup vs baseline: 1.0285x; 1.0285x over previous
import functools

import jax
import jax.numpy as jnp
from jax import lax
from jax.experimental import pallas as pl
from jax.experimental.pallas import tpu as pltpu
from jax.experimental.pallas import tpu_sc as plsc

EPS = 1e-5
Z_COEF = 0.001

S, D, H, E, F = 2048, 1024, 16, 8, 1536
DH = D // H
BM = 256
NBLK = 24
NBLK_PAD = 32
RPAD = NBLK * BM
NC, NS = 2, 16
NW = NC * NS
TPW = S // NW
D2 = D // 2


def _to_i32(a):
    return lax.bitcast_convert_type(a.reshape(a.shape[0], D2, 2), jnp.int32)


def _to_bf16(a):
    return lax.bitcast_convert_type(a, jnp.bfloat16).reshape(a.shape[0], D)


def _rmsnorm(v, w):
    return v * lax.rsqrt(jnp.mean(v * v, axis=-1, keepdims=True) + EPS) * w


def _qkv_body(x_ref, ln_ref, wq_ref, wk_ref, wv_ref, q_ref, k_ref, v_ref):
    h = _rmsnorm(x_ref[...], ln_ref[...]).astype(jnp.bfloat16)
    q_ref[...] = jnp.dot(h, wq_ref[...],
                         preferred_element_type=jnp.float32).astype(jnp.bfloat16)
    k_ref[...] = jnp.dot(h, wk_ref[...],
                         preferred_element_type=jnp.float32).astype(jnp.bfloat16)
    v_ref[...] = jnp.dot(h, wv_ref[...],
                         preferred_element_type=jnp.float32).astype(jnp.bfloat16)


def _qkv(x, ln1_w, Wq, Wk, Wv, *, bm):
    wspec = pl.BlockSpec((D, D), lambda i: (0, 0))
    mspec = pl.BlockSpec((bm, D), lambda i: (i, 0))
    return pl.pallas_call(
        _qkv_body,
        grid=(S // bm,),
        in_specs=[mspec, pl.BlockSpec((1, D), lambda i: (0, 0)),
                  wspec, wspec, wspec],
        out_specs=[mspec, mspec, mspec],
        out_shape=[jax.ShapeDtypeStruct((S, D), jnp.bfloat16)] * 3,
    )(x, ln1_w.reshape(1, D), Wq, Wk, Wv)


def _attn_body(q_ref, k_ref, v_ref, o_ref, *, scale):
    q2 = q_ref[...]
    k2 = k_ref[...]
    v2 = v_ref[...]

    outs = []
    for hh in range(2):
        sl = slice(hh * DH, (hh + 1) * DH)
        s = lax.dot_general(q2[:, sl], k2[:, sl], (((1,), (1,)), ((), ())),
                            preferred_element_type=jnp.float32) * scale
        m = jnp.max(s, axis=-1, keepdims=True)
        p = jnp.exp(s - m)
        den = jnp.sum(p, axis=-1, keepdims=True)
        o = jnp.dot(p.astype(jnp.bfloat16), v2[:, sl],
                    preferred_element_type=jnp.float32) / den
        outs.append(o)
    o_ref[...] = jnp.concatenate(outs, axis=1).astype(jnp.bfloat16)


def _attention(q, k, v, *, bq):
    return pl.pallas_call(
        functools.partial(_attn_body, scale=1.0 / (DH ** 0.5)),
        grid=(H // 2, S // bq),
        in_specs=[
            pl.BlockSpec((bq, 2 * DH), lambda g, i: (i, g)),
            pl.BlockSpec((S, 2 * DH), lambda g, i: (0, g)),
            pl.BlockSpec((S, 2 * DH), lambda g, i: (0, g)),
        ],
        out_specs=pl.BlockSpec((bq, 2 * DH), lambda g, i: (i, g)),
        out_shape=jax.ShapeDtypeStruct((S, D), jnp.bfloat16),
    )(q, k, v)


def _router_math(logits, w1_ref, w2_ref, d1_ref, d2_ref, be_ref, loss_ref):
    m = jnp.max(logits, axis=-1, keepdims=True)
    ex = jnp.exp(logits - m)
    den = jnp.sum(ex, axis=-1, keepdims=True)
    probs = ex / den

    cols = lax.broadcasted_iota(jnp.int32, (S, E), 1)
    i1 = jnp.argmax(probs, axis=-1)[:, None]
    w1 = jnp.max(probs, axis=-1, keepdims=True)
    oh1 = cols == i1
    masked = jnp.where(oh1, -jnp.inf, probs)
    i2 = jnp.argmax(masked, axis=-1)[:, None]
    w2 = jnp.max(masked, axis=-1, keepdims=True)
    oh2 = cols == i2
    tot = w1 + w2
    w1_ref[...] = w1 / tot
    w2_ref[...] = w2 / tot

    ind = jnp.logical_or(oh1, oh2).astype(jnp.bfloat16)

    C = 256
    r_io = lax.broadcasted_iota(jnp.int32, (C, C), 0)
    c_io = lax.broadcasted_iota(jnp.int32, (C, C), 1)
    tril = (c_io < r_io).astype(jnp.bfloat16)
    base = jnp.zeros((1, E), jnp.float32)
    chunks = []
    for i in range(S // C):
        ic = lax.slice(ind, (i * C, 0), ((i + 1) * C, E))
        chunks.append(jnp.dot(tril, ic, preferred_element_type=jnp.float32)
                      + base)
        base = base + jnp.sum(ic.astype(jnp.float32), axis=0, keepdims=True)
    pos = jnp.concatenate(chunks, axis=0)
    counts = base

    nbp = jnp.ceil(counts / BM) * BM
    e_r = lax.broadcasted_iota(jnp.int32, (E, E), 0)
    e_c = lax.broadcasted_iota(jnp.int32, (E, E), 1)
    triu = (e_r < e_c).astype(jnp.float32)
    off = jnp.dot(nbp, triu, preferred_element_type=jnp.float32)

    dest = off + pos
    d1_ref[...] = jnp.sum(jnp.where(oh1, dest, 0.0), axis=1,
                          keepdims=True).astype(jnp.int32)
    d2_ref[...] = jnp.sum(jnp.where(oh2, dest, 0.0), axis=1,
                          keepdims=True).astype(jnp.int32)

    bstart = lax.broadcasted_iota(jnp.int32, (NBLK_PAD, 1), 0).astype(
        jnp.float32) * BM
    inblk = jnp.logical_and(bstart >= off, bstart < off + nbp)
    ecols = lax.broadcasted_iota(jnp.int32, (NBLK_PAD, E), 1).astype(jnp.float32)
    be = jnp.sum(jnp.where(inblk, ecols + 1.0, 0.0), axis=1, keepdims=True) - 1.0
    be_ref[...] = be.astype(jnp.int32)

    z = jnp.log(den[:, 0]) + m[:, 0]
    z_loss = Z_COEF * jnp.mean(z * z)
    f = counts[0] / S
    P = jnp.mean(probs, axis=0)
    loss_ref[...] = (E * jnp.sum(f * P) + z_loss).reshape(1, 1)


def _postr_body(ao_ref, x_ref, wo_ref, ln_ref, wr_ref,
                x2_ref, h2_ref, w1_ref, w2_ref, d1_ref, d2_ref, be_ref,
                loss_ref, lg_scr, *, bm):
    i = pl.program_id(0)
    x2 = x_ref[...] + jnp.dot(ao_ref[...], wo_ref[...].astype(jnp.bfloat16),
                              preferred_element_type=jnp.float32)
    h2 = _rmsnorm(x2, ln_ref[...])
    x2_ref[...] = x2
    h2_ref[...] = h2
    lg_scr[pl.ds(i * bm, bm), :] = jnp.dot(
        h2, wr_ref[...], preferred_element_type=jnp.float32)

    @pl.when(i == S // bm - 1)
    def _():
        _router_math(lg_scr[...], w1_ref, w2_ref, d1_ref, d2_ref, be_ref,
                     loss_ref)


def _post_router(ao, x, Wo, ln2_w, Wr, *, bm):
    full = lambda i: (0, 0)
    return pl.pallas_call(
        functools.partial(_postr_body, bm=bm),
        grid=(S // bm,),
        in_specs=[
            pl.BlockSpec((bm, D), lambda i: (i, 0)),
            pl.BlockSpec((bm, D), lambda i: (i, 0)),
            pl.BlockSpec((D, D), full),
            pl.BlockSpec((1, D), full),
            pl.BlockSpec((D, E), full),
        ],
        out_specs=[
            pl.BlockSpec((bm, D), lambda i: (i, 0)),
            pl.BlockSpec((bm, D), lambda i: (i, 0)),
            pl.BlockSpec((S, 1), full),
            pl.BlockSpec((S, 1), full),
            pl.BlockSpec((S, 1), full),
            pl.BlockSpec((S, 1), full),
            pl.BlockSpec((NBLK_PAD, 1), full),
            pl.BlockSpec((1, 1), full),
        ],
        out_shape=[
            jax.ShapeDtypeStruct((S, D), jnp.float32),
            jax.ShapeDtypeStruct((S, D), jnp.float32),
            jax.ShapeDtypeStruct((S, 1), jnp.float32),
            jax.ShapeDtypeStruct((S, 1), jnp.float32),
            jax.ShapeDtypeStruct((S, 1), jnp.int32),
            jax.ShapeDtypeStruct((S, 1), jnp.int32),
            jax.ShapeDtypeStruct((NBLK_PAD, 1), jnp.int32),
            jax.ShapeDtypeStruct((1, 1), jnp.float32),
        ],
        scratch_shapes=[pltpu.VMEM((S, E), jnp.float32)],
    )(ao, x, Wo, ln2_w.reshape(1, D), Wr)


def _sc_mesh():
    return plsc.VectorSubcoreMesh(core_axis_name="c", subcore_axis_name="s")


def _sc_dispatch(h2, d1, d2):
    @functools.partial(
        pl.kernel,
        out_type=jax.ShapeDtypeStruct((RPAD, D), jnp.float32),
        mesh=_sc_mesh(),
        scratch_types=[
            pltpu.VMEM((TPW,), jnp.int32),
            pltpu.VMEM((TPW,), jnp.int32),
            pltpu.VMEM((TPW, D), jnp.float32),
            pltpu.SemaphoreType.DMA,
            pltpu.SemaphoreType.DMA,
        ],
    )
    def dispatch(h2_hbm, d1_hbm, d2_hbm, sorted_hbm,
                 idx1_v, idx2_v, rows_v, sem1, sem2):
        wid = lax.axis_index("s") * NC + lax.axis_index("c")
        base = wid * TPW
        pltpu.sync_copy(h2_hbm.at[pl.ds(base, TPW)], rows_v)
        pltpu.sync_copy(d1_hbm.at[pl.ds(base, TPW)], idx1_v)
        pltpu.sync_copy(d2_hbm.at[pl.ds(base, TPW)], idx2_v)
        cp1 = pltpu.async_copy(rows_v, sorted_hbm.at[idx1_v], sem1)
        cp2 = pltpu.async_copy(rows_v, sorted_hbm.at[idx2_v], sem2)
        cp1.wait()
        cp2.wait()

    return dispatch(h2, d1, d2)


def _moe_body(be_ref, x_ref, w1_ref, w2_ref, y_ref,
              w1b_scr, w2b_scr, prev_scr):
    b = pl.program_id(0)

    @pl.when(b == 0)
    def _():
        prev_scr[0] = -1

    be = be_ref[b]

    @pl.when(be >= 0)
    def _():
        @pl.when(prev_scr[0] != be)
        def _():
            w1b_scr[...] = w1_ref[0].astype(jnp.bfloat16)
            w2b_scr[...] = w2_ref[0].astype(jnp.bfloat16)
            prev_scr[0] = be

        xb = x_ref[...].astype(jnp.bfloat16)
        h = jnp.dot(xb, w1b_scr[...], preferred_element_type=jnp.float32)
        h = jax.nn.gelu(h).astype(jnp.bfloat16)
        y_ref[...] = jnp.dot(h, w2b_scr[...],
                             preferred_element_type=jnp.float32)


def _moe(be, sorted_x, W1, W2):
    grid_spec = pltpu.PrefetchScalarGridSpec(
        num_scalar_prefetch=1,
        grid=(NBLK,),
        in_specs=[
            pl.BlockSpec((BM, D), lambda b, be: (b, 0)),
            pl.BlockSpec((1, D, F), lambda b, be: (jnp.maximum(be[b], 0), 0, 0)),
            pl.BlockSpec((1, F, D), lambda b, be: (jnp.maximum(be[b], 0), 0, 0)),
        ],
        out_specs=pl.BlockSpec((BM, D), lambda b, be: (b, 0)),
        scratch_shapes=[
            pltpu.VMEM((D, F), jnp.bfloat16),
            pltpu.VMEM((F, D), jnp.bfloat16),
            pltpu.SMEM((1,), jnp.int32),
        ],
    )
    return pl.pallas_call(
        _moe_body,
        grid_spec=grid_spec,
        out_shape=jax.ShapeDtypeStruct((RPAD, D), jnp.float32),
    )(be, sorted_x, W1, W2)


def _sc_collect(y, d1, d2):
    @functools.partial(
        pl.kernel,
        out_type=(jax.ShapeDtypeStruct((S, D), jnp.float32),
                  jax.ShapeDtypeStruct((S, D), jnp.float32)),
        mesh=_sc_mesh(),
        scratch_types=[
            pltpu.VMEM((TPW,), jnp.int32),
            pltpu.VMEM((TPW,), jnp.int32),
            pltpu.VMEM((TPW, D), jnp.float32),
            pltpu.SemaphoreType.DMA,
        ],
    )
    def collect(y_hbm, d1_hbm, d2_hbm, y1_hbm, y2_hbm,
                idx1_v, idx2_v, rows_v, sem):
        wid = lax.axis_index("s") * NC + lax.axis_index("c")
        base = wid * TPW
        pltpu.sync_copy(d1_hbm.at[pl.ds(base, TPW)], idx1_v)
        pltpu.sync_copy(d2_hbm.at[pl.ds(base, TPW)], idx2_v)
        pltpu.async_copy(y_hbm.at[idx1_v], rows_v, sem).wait()
        pltpu.sync_copy(rows_v, y1_hbm.at[pl.ds(base, TPW)])
        pltpu.async_copy(y_hbm.at[idx2_v], rows_v, sem).wait()
        pltpu.sync_copy(rows_v, y2_hbm.at[pl.ds(base, TPW)])

    return collect(y, d1, d2)


def _combine_body(x2_ref, y1_ref, y2_ref, w1_ref, w2_ref, o_ref):
    o_ref[...] = (x2_ref[...] + w1_ref[...] * y1_ref[...]
                  + w2_ref[...] * y2_ref[...])


def _combine(x2, y1, y2, w1n, w2n, *, bm):
    return pl.pallas_call(
        _combine_body,
        grid=(S // bm,),
        in_specs=[
            pl.BlockSpec((bm, D), lambda i: (i, 0)),
            pl.BlockSpec((bm, D), lambda i: (i, 0)),
            pl.BlockSpec((bm, D), lambda i: (i, 0)),
            pl.BlockSpec((bm, 1), lambda i: (i, 0)),
            pl.BlockSpec((bm, 1), lambda i: (i, 0)),
        ],
        out_specs=pl.BlockSpec((bm, D), lambda i: (i, 0)),
        out_shape=jax.ShapeDtypeStruct((S, D), jnp.float32),
    )(x2, y1, y2, w1n, w2n)


def kernel(x, ln1_w, ln2_w, Wq, Wk, Wv, Wo, Wr, W1, W2):
    B = x.shape[0]
    xs = x.reshape(S, D)

    Wqb = Wq.astype(jnp.bfloat16)
    Wkb = Wk.astype(jnp.bfloat16)
    Wvb = Wv.astype(jnp.bfloat16)

    q, k, v = _qkv(xs, ln1_w, Wqb, Wkb, Wvb, bm=256)
    ao = _attention(q, k, v, bq=512)

    x2, h2, w1n, w2n, d1, d2, be, loss = _post_router(
        ao, xs, Wo, ln2_w, Wr, bm=256)
    d1r = d1.reshape(S)
    d2r = d2.reshape(S)

    sorted_x = _sc_dispatch(h2, d1r, d2r)
    y = _moe(be.reshape(NBLK_PAD), sorted_x, W1, W2)
    y1, y2 = _sc_collect(y, d1r, d2r)

    out = _combine(x2, y1, y2, w1n, w2n, bm=256)
    return (out.reshape(B, S, D), loss.reshape(()))


# speedup vs baseline: 1.0418x; 1.0130x over previous
import functools

import jax
import jax.numpy as jnp
from jax import lax
from jax.experimental import pallas as pl
from jax.experimental.pallas import tpu as pltpu
from jax.experimental.pallas import tpu_sc as plsc

EPS = 1e-5
Z_COEF = 0.001

S, D, H, E, F = 2048, 1024, 16, 8, 1536
DH = D // H
BM = 256
NBLK = 24
NBLK_PAD = 32
RPAD = NBLK * BM
NC, NS = 2, 16
NW = NC * NS
TPW = S // NW
D2 = D // 2


def _to_i32(a):
    return lax.bitcast_convert_type(a.reshape(a.shape[0], D2, 2), jnp.int32)


def _to_bf16(a):
    return lax.bitcast_convert_type(a, jnp.bfloat16).reshape(a.shape[0], D)


def _rmsnorm(v, w):
    return v * lax.rsqrt(jnp.mean(v * v, axis=-1, keepdims=True) + EPS) * w


def _qkv_body(x_ref, ln_ref, wq_ref, wk_ref, wv_ref, q_ref, k_ref, v_ref):
    h = _rmsnorm(x_ref[...], ln_ref[...]).astype(jnp.bfloat16)
    q_ref[...] = jnp.dot(h, wq_ref[...],
                         preferred_element_type=jnp.float32).astype(jnp.bfloat16)
    k_ref[...] = jnp.dot(h, wk_ref[...],
                         preferred_element_type=jnp.float32).astype(jnp.bfloat16)
    v_ref[...] = jnp.dot(h, wv_ref[...],
                         preferred_element_type=jnp.float32).astype(jnp.bfloat16)


def _qkv(x, ln1_w, Wq, Wk, Wv, *, bm):
    wspec = pl.BlockSpec((D, D), lambda i: (0, 0))
    mspec = pl.BlockSpec((bm, D), lambda i: (i, 0))
    return pl.pallas_call(
        _qkv_body,
        grid=(S // bm,),
        in_specs=[mspec, pl.BlockSpec((1, D), lambda i: (0, 0)),
                  wspec, wspec, wspec],
        out_specs=[mspec, mspec, mspec],
        out_shape=[jax.ShapeDtypeStruct((S, D), jnp.bfloat16)] * 3,
    )(x, ln1_w.reshape(1, D), Wq, Wk, Wv)


def _attn_body(q_ref, k_ref, v_ref, o_ref, *, scale):
    q2 = q_ref[...]
    k2 = k_ref[...]
    v2 = v_ref[...]

    outs = []
    for hh in range(2):
        sl = slice(hh * DH, (hh + 1) * DH)
        s = lax.dot_general(q2[:, sl], k2[:, sl], (((1,), (1,)), ((), ())),
                            preferred_element_type=jnp.float32) * scale
        m = jnp.max(s, axis=-1, keepdims=True)
        p = jnp.exp(s - m)
        den = jnp.sum(p, axis=-1, keepdims=True)
        o = jnp.dot(p.astype(jnp.bfloat16), v2[:, sl],
                    preferred_element_type=jnp.float32) / den
        outs.append(o)
    o_ref[...] = jnp.concatenate(outs, axis=1).astype(jnp.bfloat16)


def _attention(q, k, v, *, bq):
    return pl.pallas_call(
        functools.partial(_attn_body, scale=1.0 / (DH ** 0.5)),
        grid=(H // 2, S // bq),
        in_specs=[
            pl.BlockSpec((bq, 2 * DH), lambda g, i: (i, g)),
            pl.BlockSpec((S, 2 * DH), lambda g, i: (0, g)),
            pl.BlockSpec((S, 2 * DH), lambda g, i: (0, g)),
        ],
        out_specs=pl.BlockSpec((bq, 2 * DH), lambda g, i: (i, g)),
        out_shape=jax.ShapeDtypeStruct((S, D), jnp.bfloat16),
    )(q, k, v)


def _router_math(logits, w1_ref, w2_ref, d1_ref, d2_ref, be_ref, loss_ref):
    m = jnp.max(logits, axis=-1, keepdims=True)
    ex = jnp.exp(logits - m)
    den = jnp.sum(ex, axis=-1, keepdims=True)
    probs = ex / den

    cols = lax.broadcasted_iota(jnp.int32, (S, E), 1)
    i1 = jnp.argmax(probs, axis=-1)[:, None]
    w1 = jnp.max(probs, axis=-1, keepdims=True)
    oh1 = cols == i1
    masked = jnp.where(oh1, -jnp.inf, probs)
    i2 = jnp.argmax(masked, axis=-1)[:, None]
    w2 = jnp.max(masked, axis=-1, keepdims=True)
    oh2 = cols == i2
    tot = w1 + w2
    w1_ref[...] = w1 / tot
    w2_ref[...] = w2 / tot

    ind = jnp.logical_or(oh1, oh2).astype(jnp.bfloat16)

    C = 256
    r_io = lax.broadcasted_iota(jnp.int32, (C, C), 0)
    c_io = lax.broadcasted_iota(jnp.int32, (C, C), 1)
    tril = (c_io < r_io).astype(jnp.bfloat16)
    base = jnp.zeros((1, E), jnp.float32)
    chunks = []
    for i in range(S // C):
        ic = lax.slice(ind, (i * C, 0), ((i + 1) * C, E))
        chunks.append(jnp.dot(tril, ic, preferred_element_type=jnp.float32)
                      + base)
        base = base + jnp.sum(ic.astype(jnp.float32), axis=0, keepdims=True)
    pos = jnp.concatenate(chunks, axis=0)
    counts = base

    nbp = jnp.ceil(counts / BM) * BM
    e_r = lax.broadcasted_iota(jnp.int32, (E, E), 0)
    e_c = lax.broadcasted_iota(jnp.int32, (E, E), 1)
    triu = (e_r < e_c).astype(jnp.float32)
    off = jnp.dot(nbp, triu, preferred_element_type=jnp.float32)

    dest = off + pos
    d1_ref[...] = jnp.sum(jnp.where(oh1, dest, 0.0), axis=1,
                          keepdims=True).astype(jnp.int32)
    d2_ref[...] = jnp.sum(jnp.where(oh2, dest, 0.0), axis=1,
                          keepdims=True).astype(jnp.int32)

    bstart = lax.broadcasted_iota(jnp.int32, (NBLK_PAD, 1), 0).astype(
        jnp.float32) * BM
    inblk = jnp.logical_and(bstart >= off, bstart < off + nbp)
    ecols = lax.broadcasted_iota(jnp.int32, (NBLK_PAD, E), 1).astype(jnp.float32)
    be = jnp.sum(jnp.where(inblk, ecols + 1.0, 0.0), axis=1, keepdims=True) - 1.0
    be_ref[...] = be.astype(jnp.int32)

    z = jnp.log(den[:, 0]) + m[:, 0]
    z_loss = Z_COEF * jnp.mean(z * z)
    f = counts[0] / S
    P = jnp.mean(probs, axis=0)
    loss_ref[...] = (E * jnp.sum(f * P) + z_loss).reshape(1, 1)


def _postr_body(ao_ref, x_ref, wo_ref, ln_ref, wr_ref,
                x2_ref, h2_ref, w1_ref, w2_ref, d1_ref, d2_ref, be_ref,
                loss_ref, lg_scr, *, bm):
    i = pl.program_id(0)
    x2 = x_ref[...] + jnp.dot(ao_ref[...], wo_ref[...].astype(jnp.bfloat16),
                              preferred_element_type=jnp.float32)
    h2 = _rmsnorm(x2, ln_ref[...])
    x2_ref[...] = x2
    h2_ref[...] = h2
    lg_scr[pl.ds(i * bm, bm), :] = jnp.dot(
        h2, wr_ref[...], preferred_element_type=jnp.float32)

    @pl.when(i == S // bm - 1)
    def _():
        _router_math(lg_scr[...], w1_ref, w2_ref, d1_ref, d2_ref, be_ref,
                     loss_ref)


def _post_router(ao, x, Wo, ln2_w, Wr, *, bm):
    full = lambda i: (0, 0)
    return pl.pallas_call(
        functools.partial(_postr_body, bm=bm),
        grid=(S // bm,),
        in_specs=[
            pl.BlockSpec((bm, D), lambda i: (i, 0)),
            pl.BlockSpec((bm, D), lambda i: (i, 0)),
            pl.BlockSpec((D, D), full),
            pl.BlockSpec((1, D), full),
            pl.BlockSpec((D, E), full),
        ],
        out_specs=[
            pl.BlockSpec((bm, D), lambda i: (i, 0)),
            pl.BlockSpec((bm, D), lambda i: (i, 0)),
            pl.BlockSpec((S, 1), full),
            pl.BlockSpec((S, 1), full),
            pl.BlockSpec((S, 1), full),
            pl.BlockSpec((S, 1), full),
            pl.BlockSpec((NBLK_PAD, 1), full),
            pl.BlockSpec((1, 1), full),
        ],
        out_shape=[
            jax.ShapeDtypeStruct((S, D), jnp.float32),
            jax.ShapeDtypeStruct((S, D), jnp.float32),
            jax.ShapeDtypeStruct((S, 1), jnp.float32),
            jax.ShapeDtypeStruct((S, 1), jnp.float32),
            jax.ShapeDtypeStruct((S, 1), jnp.int32),
            jax.ShapeDtypeStruct((S, 1), jnp.int32),
            jax.ShapeDtypeStruct((NBLK_PAD, 1), jnp.int32),
            jax.ShapeDtypeStruct((1, 1), jnp.float32),
        ],
        scratch_shapes=[pltpu.VMEM((S, E), jnp.float32)],
    )(ao, x, Wo, ln2_w.reshape(1, D), Wr)


def _sc_mesh():
    return plsc.VectorSubcoreMesh(core_axis_name="c", subcore_axis_name="s")


def _sc_dispatch(h2, d1, d2):
    @functools.partial(
        pl.kernel,
        out_type=jax.ShapeDtypeStruct((RPAD, D), jnp.float32),
        mesh=_sc_mesh(),
        scratch_types=[
            pltpu.VMEM((TPW,), jnp.int32),
            pltpu.VMEM((TPW,), jnp.int32),
            pltpu.VMEM((TPW, D), jnp.float32),
            pltpu.SemaphoreType.DMA,
            pltpu.SemaphoreType.DMA,
        ],
    )
    def dispatch(h2_hbm, d1_hbm, d2_hbm, sorted_hbm,
                 idx1_v, idx2_v, rows_v, sem1, sem2):
        wid = lax.axis_index("s") * NC + lax.axis_index("c")
        base = wid * TPW
        pltpu.sync_copy(h2_hbm.at[pl.ds(base, TPW)], rows_v)
        pltpu.sync_copy(d1_hbm.at[pl.ds(base, TPW)], idx1_v)
        pltpu.sync_copy(d2_hbm.at[pl.ds(base, TPW)], idx2_v)
        cp1 = pltpu.async_copy(rows_v, sorted_hbm.at[idx1_v], sem1)
        cp2 = pltpu.async_copy(rows_v, sorted_hbm.at[idx2_v], sem2)
        cp1.wait()
        cp2.wait()

    return dispatch(h2, d1, d2)


def _moe_body(be_ref, x_ref, w1_ref, w2_ref, y_ref):
    b = pl.program_id(0)

    @pl.when(be_ref[b] >= 0)
    def _():
        xb = x_ref[...].astype(jnp.bfloat16)
        CF = 512
        hs = []
        for f0 in range(0, F, CF):
            w1c = w1_ref[0, :, f0:f0 + CF].astype(jnp.bfloat16)
            hs.append(jnp.dot(xb, w1c, preferred_element_type=jnp.float32))
        hx = jax.nn.gelu(jnp.concatenate(hs, axis=1)).astype(jnp.bfloat16)
        acc = jnp.zeros((BM, D), jnp.float32)
        for f0 in range(0, F, CF):
            w2c = w2_ref[0, f0:f0 + CF, :].astype(jnp.bfloat16)
            acc = acc + jnp.dot(hx[:, f0:f0 + CF], w2c,
                                preferred_element_type=jnp.float32)
        y_ref[...] = acc


def _moe(be, sorted_x, W1, W2):
    grid_spec = pltpu.PrefetchScalarGridSpec(
        num_scalar_prefetch=1,
        grid=(NBLK,),
        in_specs=[
            pl.BlockSpec((BM, D), lambda b, be: (b, 0)),
            pl.BlockSpec((1, D, F), lambda b, be: (jnp.maximum(be[b], 0), 0, 0)),
            pl.BlockSpec((1, F, D), lambda b, be: (jnp.maximum(be[b], 0), 0, 0)),
        ],
        out_specs=pl.BlockSpec((BM, D), lambda b, be: (b, 0)),
    )
    return pl.pallas_call(
        _moe_body,
        grid_spec=grid_spec,
        out_shape=jax.ShapeDtypeStruct((RPAD, D), jnp.float32),
    )(be, sorted_x, W1, W2)


def _sc_collect(y, d1, d2):
    @functools.partial(
        pl.kernel,
        out_type=(jax.ShapeDtypeStruct((S, D), jnp.float32),
                  jax.ShapeDtypeStruct((S, D), jnp.float32)),
        mesh=_sc_mesh(),
        scratch_types=[
            pltpu.VMEM((TPW,), jnp.int32),
            pltpu.VMEM((TPW,), jnp.int32),
            pltpu.VMEM((TPW, D), jnp.float32),
            pltpu.SemaphoreType.DMA,
        ],
    )
    def collect(y_hbm, d1_hbm, d2_hbm, y1_hbm, y2_hbm,
                idx1_v, idx2_v, rows_v, sem):
        wid = lax.axis_index("s") * NC + lax.axis_index("c")
        base = wid * TPW
        pltpu.sync_copy(d1_hbm.at[pl.ds(base, TPW)], idx1_v)
        pltpu.sync_copy(d2_hbm.at[pl.ds(base, TPW)], idx2_v)
        pltpu.async_copy(y_hbm.at[idx1_v], rows_v, sem).wait()
        pltpu.sync_copy(rows_v, y1_hbm.at[pl.ds(base, TPW)])
        pltpu.async_copy(y_hbm.at[idx2_v], rows_v, sem).wait()
        pltpu.sync_copy(rows_v, y2_hbm.at[pl.ds(base, TPW)])

    return collect(y, d1, d2)


def _combine_body(x2_ref, y1_ref, y2_ref, w1_ref, w2_ref, o_ref):
    o_ref[...] = (x2_ref[...] + w1_ref[...] * y1_ref[...]
                  + w2_ref[...] * y2_ref[...])


def _combine(x2, y1, y2, w1n, w2n, *, bm):
    return pl.pallas_call(
        _combine_body,
        grid=(S // bm,),
        in_specs=[
            pl.BlockSpec((bm, D), lambda i: (i, 0)),
            pl.BlockSpec((bm, D), lambda i: (i, 0)),
            pl.BlockSpec((bm, D), lambda i: (i, 0)),
            pl.BlockSpec((bm, 1), lambda i: (i, 0)),
            pl.BlockSpec((bm, 1), lambda i: (i, 0)),
        ],
        out_specs=pl.BlockSpec((bm, D), lambda i: (i, 0)),
        out_shape=jax.ShapeDtypeStruct((S, D), jnp.float32),
    )(x2, y1, y2, w1n, w2n)


def kernel(x, ln1_w, ln2_w, Wq, Wk, Wv, Wo, Wr, W1, W2):
    B = x.shape[0]
    xs = x.reshape(S, D)

    Wqb = Wq.astype(jnp.bfloat16)
    Wkb = Wk.astype(jnp.bfloat16)
    Wvb = Wv.astype(jnp.bfloat16)

    q, k, v = _qkv(xs, ln1_w, Wqb, Wkb, Wvb, bm=256)
    ao = _attention(q, k, v, bq=512)

    x2, h2, w1n, w2n, d1, d2, be, loss = _post_router(
        ao, xs, Wo, ln2_w, Wr, bm=256)
    d1r = d1.reshape(S)
    d2r = d2.reshape(S)

    sorted_x = _sc_dispatch(h2, d1r, d2r)
    y = _moe(be.reshape(NBLK_PAD), sorted_x, W1, W2)
    y1, y2 = _sc_collect(y, d1r, d2r)

    out = _combine(x2, y1, y2, w1n, w2n, bm=256)
    return (out.reshape(B, S, D), loss.reshape(()))


# speedup vs baseline: 1.1451x; 1.0992x over previous
import functools

import jax
import jax.numpy as jnp
from jax import lax
from jax.experimental import pallas as pl
from jax.experimental.pallas import tpu as pltpu
from jax.experimental.pallas import tpu_sc as plsc

EPS = 1e-5
Z_COEF = 0.001

S, D, H, E, F = 2048, 1024, 16, 8, 1536
DH = D // H
BM = 256
NBLK = 24
NBLK_PAD = 32
RPAD = NBLK * BM
NC, NS = 2, 16
NW = NC * NS
TPW = S // NW
D2 = D // 2


def _to_i32(a):
    return lax.bitcast_convert_type(a.reshape(a.shape[0], D2, 2), jnp.int32)


def _to_bf16(a):
    return lax.bitcast_convert_type(a, jnp.bfloat16).reshape(a.shape[0], D)


def _rmsnorm(v, w):
    return v * lax.rsqrt(jnp.mean(v * v, axis=-1, keepdims=True) + EPS) * w


def _qkv_body(x_ref, ln_ref, wq_ref, wk_ref, wv_ref, q_ref, k_ref, v_ref):
    h = _rmsnorm(x_ref[...], ln_ref[...]).astype(jnp.bfloat16)
    q_ref[...] = jnp.dot(h, wq_ref[...],
                         preferred_element_type=jnp.float32).astype(jnp.bfloat16)
    k_ref[...] = jnp.dot(h, wk_ref[...],
                         preferred_element_type=jnp.float32).astype(jnp.bfloat16)
    v_ref[...] = jnp.dot(h, wv_ref[...],
                         preferred_element_type=jnp.float32).astype(jnp.bfloat16)


def _qkv(x, ln1_w, Wq, Wk, Wv, *, bm):
    wspec = pl.BlockSpec((D, D), lambda i: (0, 0))
    mspec = pl.BlockSpec((bm, D), lambda i: (i, 0))
    return pl.pallas_call(
        _qkv_body,
        grid=(S // bm,),
        in_specs=[mspec, pl.BlockSpec((1, D), lambda i: (0, 0)),
                  wspec, wspec, wspec],
        out_specs=[mspec, mspec, mspec],
        out_shape=[jax.ShapeDtypeStruct((S, D), jnp.bfloat16)] * 3,
    )(x, ln1_w.reshape(1, D), Wq, Wk, Wv)


def _attn_body(q_ref, k_ref, v_ref, o_ref, *, scale):
    q2 = q_ref[...]
    k2 = k_ref[...]
    v2 = v_ref[...]

    def one(qh, kh):
        s = lax.dot_general(qh, kh, (((1,), (1,)), ((), ())),
                            preferred_element_type=jnp.float32) * scale
        m = jnp.max(s, axis=-1, keepdims=True)
        p = jnp.exp(s - m)
        den = jnp.sum(p, axis=-1, keepdims=True)
        return p.astype(jnp.bfloat16), den

    p_a, den_a = one(q2[:, :DH], k2[:, :DH])
    p_b, den_b = one(q2[:, DH:], k2[:, DH:])
    o_a = jnp.dot(p_a, v2[:, :DH], preferred_element_type=jnp.float32) / den_a
    o_b = jnp.dot(p_b, v2[:, DH:], preferred_element_type=jnp.float32) / den_b
    o_ref[...] = jnp.concatenate([o_a, o_b], axis=1).astype(jnp.bfloat16)


def _attention(q, k, v, *, bq):
    return pl.pallas_call(
        functools.partial(_attn_body, scale=1.0 / (DH ** 0.5)),
        grid=(H // 2, S // bq),
        in_specs=[
            pl.BlockSpec((bq, 2 * DH), lambda g, i: (i, g)),
            pl.BlockSpec((S, 2 * DH), lambda g, i: (0, g)),
            pl.BlockSpec((S, 2 * DH), lambda g, i: (0, g)),
        ],
        out_specs=pl.BlockSpec((bq, 2 * DH), lambda g, i: (i, g)),
        out_shape=jax.ShapeDtypeStruct((S, D), jnp.bfloat16),
    )(q, k, v)


def _router_math(logits, w1_ref, w2_ref, d1_ref, d2_ref, be_ref, loss_ref):
    m = jnp.max(logits, axis=-1, keepdims=True)
    ex = jnp.exp(logits - m)
    den = jnp.sum(ex, axis=-1, keepdims=True)
    probs = ex / den

    cols = lax.broadcasted_iota(jnp.int32, (S, E), 1)
    i1 = jnp.argmax(probs, axis=-1)[:, None]
    w1 = jnp.max(probs, axis=-1, keepdims=True)
    oh1 = cols == i1
    masked = jnp.where(oh1, -jnp.inf, probs)
    i2 = jnp.argmax(masked, axis=-1)[:, None]
    w2 = jnp.max(masked, axis=-1, keepdims=True)
    oh2 = cols == i2
    tot = w1 + w2
    w1_ref[...] = w1 / tot
    w2_ref[...] = w2 / tot

    ind = jnp.logical_or(oh1, oh2).astype(jnp.bfloat16)

    C = 256
    r_io = lax.broadcasted_iota(jnp.int32, (C, C), 0)
    c_io = lax.broadcasted_iota(jnp.int32, (C, C), 1)
    tril = (c_io < r_io).astype(jnp.bfloat16)
    base = jnp.zeros((1, E), jnp.float32)
    chunks = []
    for i in range(S // C):
        ic = lax.slice(ind, (i * C, 0), ((i + 1) * C, E))
        chunks.append(jnp.dot(tril, ic, preferred_element_type=jnp.float32)
                      + base)
        base = base + jnp.sum(ic.astype(jnp.float32), axis=0, keepdims=True)
    pos = jnp.concatenate(chunks, axis=0)
    counts = base

    nbp = jnp.ceil(counts / BM) * BM
    e_r = lax.broadcasted_iota(jnp.int32, (E, E), 0)
    e_c = lax.broadcasted_iota(jnp.int32, (E, E), 1)
    triu = (e_r < e_c).astype(jnp.float32)
    off = jnp.dot(nbp, triu, preferred_element_type=jnp.float32)

    dest = off + pos
    d1_ref[...] = jnp.sum(jnp.where(oh1, dest, 0.0), axis=1,
                          keepdims=True).astype(jnp.int32)
    d2_ref[...] = jnp.sum(jnp.where(oh2, dest, 0.0), axis=1,
                          keepdims=True).astype(jnp.int32)

    bstart = lax.broadcasted_iota(jnp.int32, (NBLK_PAD, 1), 0).astype(
        jnp.float32) * BM
    inblk = jnp.logical_and(bstart >= off, bstart < off + nbp)
    ecols = lax.broadcasted_iota(jnp.int32, (NBLK_PAD, E), 1).astype(jnp.float32)
    be = jnp.sum(jnp.where(inblk, ecols + 1.0, 0.0), axis=1, keepdims=True) - 1.0
    be_ref[...] = be.astype(jnp.int32)

    z = jnp.log(den[:, 0]) + m[:, 0]
    z_loss = Z_COEF * jnp.mean(z * z)
    f = counts[0] / S
    P = jnp.mean(probs, axis=0)
    loss_ref[...] = (E * jnp.sum(f * P) + z_loss).reshape(1, 1)


def _postr_body(ao_ref, x_ref, wo_ref, ln_ref, wr_ref,
                x2_ref, h2_ref, w1_ref, w2_ref, d1_ref, d2_ref, be_ref,
                loss_ref, lg_scr, *, bm):
    i = pl.program_id(0)
    x2 = x_ref[...] + jnp.dot(ao_ref[...], wo_ref[...].astype(jnp.bfloat16),
                              preferred_element_type=jnp.float32)
    h2 = _rmsnorm(x2, ln_ref[...])
    x2_ref[...] = x2
    h2_ref[...] = h2
    lg_scr[pl.ds(i * bm, bm), :] = jnp.dot(
        h2, wr_ref[...], preferred_element_type=jnp.float32)

    @pl.when(i == S // bm - 1)
    def _():
        _router_math(lg_scr[...], w1_ref, w2_ref, d1_ref, d2_ref, be_ref,
                     loss_ref)


def _post_router(ao, x, Wo, ln2_w, Wr, *, bm):
    full = lambda i: (0, 0)
    return pl.pallas_call(
        functools.partial(_postr_body, bm=bm),
        grid=(S // bm,),
        in_specs=[
            pl.BlockSpec((bm, D), lambda i: (i, 0)),
            pl.BlockSpec((bm, D), lambda i: (i, 0)),
            pl.BlockSpec((D, D), full),
            pl.BlockSpec((1, D), full),
            pl.BlockSpec((D, E), full),
        ],
        out_specs=[
            pl.BlockSpec((bm, D), lambda i: (i, 0)),
            pl.BlockSpec((bm, D), lambda i: (i, 0)),
            pl.BlockSpec((S, 1), full),
            pl.BlockSpec((S, 1), full),
            pl.BlockSpec((S, 1), full),
            pl.BlockSpec((S, 1), full),
            pl.BlockSpec((NBLK_PAD, 1), full),
            pl.BlockSpec((1, 1), full),
        ],
        out_shape=[
            jax.ShapeDtypeStruct((S, D), jnp.float32),
            jax.ShapeDtypeStruct((S, D), jnp.float32),
            jax.ShapeDtypeStruct((S, 1), jnp.float32),
            jax.ShapeDtypeStruct((S, 1), jnp.float32),
            jax.ShapeDtypeStruct((S, 1), jnp.int32),
            jax.ShapeDtypeStruct((S, 1), jnp.int32),
            jax.ShapeDtypeStruct((NBLK_PAD, 1), jnp.int32),
            jax.ShapeDtypeStruct((1, 1), jnp.float32),
        ],
        scratch_shapes=[pltpu.VMEM((S, E), jnp.float32)],
    )(ao, x, Wo, ln2_w.reshape(1, D), Wr)


def _sc_mesh():
    return plsc.VectorSubcoreMesh(core_axis_name="c", subcore_axis_name="s")


def _sc_dispatch(h2, d1, d2):
    @functools.partial(
        pl.kernel,
        out_type=jax.ShapeDtypeStruct((RPAD, D), jnp.float32),
        mesh=_sc_mesh(),
        scratch_types=[
            pltpu.VMEM((TPW,), jnp.int32),
            pltpu.VMEM((TPW,), jnp.int32),
            pltpu.VMEM((TPW, D), jnp.float32),
            pltpu.SemaphoreType.DMA,
            pltpu.SemaphoreType.DMA,
        ],
    )
    def dispatch(h2_hbm, d1_hbm, d2_hbm, sorted_hbm,
                 idx1_v, idx2_v, rows_v, sem1, sem2):
        wid = lax.axis_index("s") * NC + lax.axis_index("c")
        base = wid * TPW
        pltpu.sync_copy(h2_hbm.at[pl.ds(base, TPW)], rows_v)
        pltpu.sync_copy(d1_hbm.at[pl.ds(base, TPW)], idx1_v)
        pltpu.sync_copy(d2_hbm.at[pl.ds(base, TPW)], idx2_v)
        cp1 = pltpu.async_copy(rows_v, sorted_hbm.at[idx1_v], sem1)
        cp2 = pltpu.async_copy(rows_v, sorted_hbm.at[idx2_v], sem2)
        cp1.wait()
        cp2.wait()

    return dispatch(h2, d1, d2)


def _moe_body(be_ref, x_ref, w1_ref, w2_ref, y_ref):
    b = pl.program_id(0)

    @pl.when(be_ref[b] >= 0)
    def _():
        xb = x_ref[...].astype(jnp.bfloat16)
        CF = 512
        hs = []
        for f0 in range(0, F, CF):
            w1c = w1_ref[0, :, f0:f0 + CF].astype(jnp.bfloat16)
            hs.append(jnp.dot(xb, w1c, preferred_element_type=jnp.float32))
        hx = jax.nn.gelu(jnp.concatenate(hs, axis=1)).astype(jnp.bfloat16)
        acc = jnp.zeros((BM, D), jnp.float32)
        for f0 in range(0, F, CF):
            w2c = w2_ref[0, f0:f0 + CF, :].astype(jnp.bfloat16)
            acc = acc + jnp.dot(hx[:, f0:f0 + CF], w2c,
                                preferred_element_type=jnp.float32)
        y_ref[...] = acc


def _moe(be, sorted_x, W1, W2):
    grid_spec = pltpu.PrefetchScalarGridSpec(
        num_scalar_prefetch=1,
        grid=(NBLK,),
        in_specs=[
            pl.BlockSpec((BM, D), lambda b, be: (b, 0)),
            pl.BlockSpec((1, D, F), lambda b, be: (jnp.maximum(be[b], 0), 0, 0)),
            pl.BlockSpec((1, F, D), lambda b, be: (jnp.maximum(be[b], 0), 0, 0)),
        ],
        out_specs=pl.BlockSpec((BM, D), lambda b, be: (b, 0)),
    )
    return pl.pallas_call(
        _moe_body,
        grid_spec=grid_spec,
        out_shape=jax.ShapeDtypeStruct((RPAD, D), jnp.float32),
    )(be, sorted_x, W1, W2)


def _sc_collect(y, d1, d2):
    @functools.partial(
        pl.kernel,
        out_type=(jax.ShapeDtypeStruct((S, D), jnp.float32),
                  jax.ShapeDtypeStruct((S, D), jnp.float32)),
        mesh=_sc_mesh(),
        scratch_types=[
            pltpu.VMEM((TPW,), jnp.int32),
            pltpu.VMEM((TPW,), jnp.int32),
            pltpu.VMEM((TPW, D), jnp.float32),
            pltpu.SemaphoreType.DMA,
        ],
    )
    def collect(y_hbm, d1_hbm, d2_hbm, y1_hbm, y2_hbm,
                idx1_v, idx2_v, rows_v, sem):
        wid = lax.axis_index("s") * NC + lax.axis_index("c")
        base = wid * TPW
        pltpu.sync_copy(d1_hbm.at[pl.ds(base, TPW)], idx1_v)
        pltpu.sync_copy(d2_hbm.at[pl.ds(base, TPW)], idx2_v)
        pltpu.async_copy(y_hbm.at[idx1_v], rows_v, sem).wait()
        pltpu.sync_copy(rows_v, y1_hbm.at[pl.ds(base, TPW)])
        pltpu.async_copy(y_hbm.at[idx2_v], rows_v, sem).wait()
        pltpu.sync_copy(rows_v, y2_hbm.at[pl.ds(base, TPW)])

    return collect(y, d1, d2)


def _combine_body(x2_ref, y1_ref, y2_ref, w1_ref, w2_ref, o_ref):
    o_ref[...] = (x2_ref[...] + w1_ref[...] * y1_ref[...]
                  + w2_ref[...] * y2_ref[...])


def _combine(x2, y1, y2, w1n, w2n, *, bm):
    return pl.pallas_call(
        _combine_body,
        grid=(S // bm,),
        in_specs=[
            pl.BlockSpec((bm, D), lambda i: (i, 0)),
            pl.BlockSpec((bm, D), lambda i: (i, 0)),
            pl.BlockSpec((bm, D), lambda i: (i, 0)),
            pl.BlockSpec((bm, 1), lambda i: (i, 0)),
            pl.BlockSpec((bm, 1), lambda i: (i, 0)),
        ],
        out_specs=pl.BlockSpec((bm, D), lambda i: (i, 0)),
        out_shape=jax.ShapeDtypeStruct((S, D), jnp.float32),
    )(x2, y1, y2, w1n, w2n)


def kernel(x, ln1_w, ln2_w, Wq, Wk, Wv, Wo, Wr, W1, W2):
    B = x.shape[0]
    xs = x.reshape(S, D)

    Wqb = Wq.astype(jnp.bfloat16)
    Wkb = Wk.astype(jnp.bfloat16)
    Wvb = Wv.astype(jnp.bfloat16)

    q, k, v = _qkv(xs, ln1_w, Wqb, Wkb, Wvb, bm=256)
    ao = _attention(q, k, v, bq=512)

    x2, h2, w1n, w2n, d1, d2, be, loss = _post_router(
        ao, xs, Wo, ln2_w, Wr, bm=256)
    d1r = d1.reshape(S)
    d2r = d2.reshape(S)

    sorted_x = _sc_dispatch(h2, d1r, d2r)
    y = _moe(be.reshape(NBLK_PAD), sorted_x, W1, W2)
    y1, y2 = _sc_collect(y, d1r, d2r)

    out = _combine(x2, y1, y2, w1n, w2n, bm=256)
    return (out.reshape(B, S, D), loss.reshape(()))
